# baseline (speedup 1.0000x reference)
import jax
import jax.numpy as jnp
from jax import lax
from jax.experimental import pallas as pl
from jax.experimental.pallas import tpu as pltpu
from jax.experimental.pallas import tpu_sc as plsc

N_ROWS = 100000
D = 128
NSEG = 64
CHUNK = 128
NC, NS = 2, 16
NW = NC * NS
N_FULL = 468
N_SC = N_FULL * CHUNK
MAXC = -(-N_FULL // NW)
HI = N_FULL - (MAXC - 1) * NW
NBUF = 6

TC_B = 512
TC_BLK0 = N_SC // TC_B
N_TC = N_ROWS - N_SC
TC_NB = -(-N_TC // TC_B)


def _body(nodes, ids, zeros, out, ibufs, bufs, acc_sh, semns, semis, semscs):
    c = lax.axis_index("c")
    s = lax.axis_index("s")
    wid = s * NC + c

    def gather(j):
        b = j % NBUF
        r0 = (wid + j * NW) * CHUNK
        pltpu.async_copy(nodes.at[pl.ds(r0, CHUNK)], bufs[b], semns[b])
        pltpu.async_copy(ids.at[pl.ds(r0, CHUNK)], ibufs[b], semis[b])

    def gather_wait(j):
        b = j % NBUF
        pltpu.make_async_copy(nodes.at[pl.ds(0, CHUNK)], bufs[b], semns[b]).wait()
        pltpu.make_async_copy(ids.at[pl.ds(0, CHUNK)], ibufs[b], semis[b]).wait()

    def scatter(j):
        b = j % NBUF
        pltpu.async_copy(bufs[b], acc_sh.at[ibufs[b]], semscs[b], add=True)

    def scatter_wait(j):
        b = j % NBUF
        pltpu.make_async_copy(bufs[b], acc_sh.at[ibufs[b]], semscs[b]).wait()

    for j0 in range(NBUF - 2):
        gather(j0)

    @pl.when(s == 0)
    def _init():
        pltpu.sync_copy(zeros, acc_sh)

    plsc.subcore_barrier()

    for i in range(MAXC):
        if i >= 2:
            scatter_wait(i - 2)

        def step(i=i):
            j = i + NBUF - 2
            if j < MAXC:
                if j == MAXC - 1:
                    @pl.when(wid < HI)
                    def _():
                        gather(j)
                else:
                    gather(j)
            gather_wait(i)
            scatter(i)

        if i == MAXC - 1:
            @pl.when(wid < HI)
            def _():
                step()
        else:
            step()

    scatter_wait(MAXC - 2)

    @pl.when(wid < HI)
    def _last():
        scatter_wait(MAXC - 1)

    plsc.subcore_barrier()

    @pl.when(s == 0)
    def _flush():
        pltpu.sync_copy(acc_sh, out.at[c])


def _tc_body(ids_ref, nodes_ref, o_ref):
    i = pl.program_id(0)
    blk = nodes_ref[...]
    rows_d = i * TC_B + lax.broadcasted_iota(jnp.int32, (TC_B, D), 0)
    blk = jnp.where(rows_d < N_TC, blk, 0.0)
    idsb = ids_ref[0, 0]
    rows = i * TC_B + lax.broadcasted_iota(jnp.int32, (TC_B, NSEG), 0)
    segs = lax.broadcasted_iota(jnp.int32, (TC_B, NSEG), 1)
    onehot = jnp.where((idsb[:, None] == segs) & (rows < N_TC), 1.0, 0.0)
    contrib = lax.dot_general(onehot, blk, (((0,), (0,)), ((), ())),
                              preferred_element_type=jnp.float32)

    @pl.when(i == 0)
    def _first():
        o_ref[...] = contrib

    @pl.when(i > 0)
    def _rest():
        o_ref[...] += contrib


@jax.jit
def _segsum(nodes, ids32, zeros):
    mesh = plsc.VectorSubcoreMesh(core_axis_name="c", subcore_axis_name="s")
    sc_partials = pl.kernel(
        _body,
        out_type=jax.ShapeDtypeStruct((NC, NSEG, D), jnp.float32),
        mesh=mesh,
        scratch_types=[
            [pltpu.VMEM((CHUNK,), jnp.int32) for _ in range(NBUF)],
            [pltpu.VMEM((CHUNK, D), jnp.float32) for _ in range(NBUF)],
            pltpu.VMEM_SHARED((NSEG, D), jnp.float32),
            [pltpu.SemaphoreType.DMA for _ in range(NBUF)],
            [pltpu.SemaphoreType.DMA for _ in range(NBUF)],
            [pltpu.SemaphoreType.DMA for _ in range(NBUF)],
        ],
    )(nodes, ids32, zeros)

    ids_tc = jnp.pad(ids32[N_SC:], (0, TC_NB * TC_B - N_TC)).reshape(TC_NB, 1, TC_B)
    tc_out = pl.pallas_call(
        _tc_body,
        grid=(TC_NB,),
        in_specs=[
            pl.BlockSpec((1, 1, TC_B), lambda i: (i, 0, 0)),
            pl.BlockSpec((TC_B, D), lambda i: (TC_BLK0 + i, 0)),
        ],
        out_specs=pl.BlockSpec((NSEG, D), lambda i: (0, 0)),
        out_shape=jax.ShapeDtypeStruct((NSEG, D), jnp.float32),
    )(ids_tc, nodes)

    return sc_partials[0] + sc_partials[1] + tc_out


def kernel(nodes, segment_ids, num_graphs):
    del num_graphs
    ids32 = segment_ids.astype(jnp.int32)
    zeros = jnp.zeros((NSEG, D), jnp.float32)
    return _segsum(nodes, ids32, zeros)


# speedup vs baseline: 1.4442x; 1.4442x over previous
import jax
import jax.numpy as jnp
from jax import lax
from jax.experimental import pallas as pl
from jax.experimental.pallas import tpu as pltpu
from jax.experimental.pallas import tpu_sc as plsc

N_ROWS = 100000
D = 128
NSEG = 64
CHUNK = 128
N_FULL = N_ROWS // CHUNK
TAIL = N_ROWS - N_FULL * CHUNK
NC, NS = 2, 16
NW = NC * NS
MAXC = -(-N_FULL // NW)
HI = N_FULL - (MAXC - 1) * NW
NBUF = 6


def _body(nodes, ids, zeros, out,
          ibufs, bufs, tidx_v, tail_v, acc_sh, semns, semis, semscs, sem_t):
    c = lax.axis_index("c")
    s = lax.axis_index("s")
    wid = s * NC + c

    def gather(j):
        b = j % NBUF
        r0 = (wid + j * NW) * CHUNK
        pltpu.async_copy(nodes.at[pl.ds(r0, CHUNK)], bufs[b], semns[b])
        pltpu.async_copy(ids.at[pl.ds(r0, CHUNK)], ibufs[b], semis[b])

    def gather_wait(j):
        b = j % NBUF
        pltpu.make_async_copy(nodes.at[pl.ds(0, CHUNK)], bufs[b], semns[b]).wait()
        pltpu.make_async_copy(ids.at[pl.ds(0, CHUNK)], ibufs[b], semis[b]).wait()

    def scatter(j):
        b = j % NBUF
        pltpu.async_copy(bufs[b], acc_sh.at[ibufs[b]], semscs[b], add=True)

    def scatter_wait(j):
        b = j % NBUF
        pltpu.make_async_copy(bufs[b], acc_sh.at[ibufs[b]], semscs[b]).wait()

    for j0 in range(NBUF - 2):
        gather(j0)

    @pl.when(s == 0)
    def _init():
        pltpu.sync_copy(zeros, acc_sh)

    plsc.subcore_barrier()

    for i in range(MAXC):
        if i >= 2:
            scatter_wait(i - 2)

        def step(i=i):
            j = i + NBUF - 2
            if j < MAXC:
                if j == MAXC - 1:
                    @pl.when(wid < HI)
                    def _():
                        gather(j)
                else:
                    gather(j)
            gather_wait(i)
            scatter(i)

        if i == MAXC - 1:
            @pl.when(wid < HI)
            def _():
                step()
        else:
            step()

    scatter_wait(MAXC - 2)

    @pl.when(wid < HI)
    def _last():
        scatter_wait(MAXC - 1)

    @pl.when(wid == NW - 1)
    def _tail():
        r0 = N_FULL * CHUNK
        pltpu.sync_copy(ids.at[pl.ds(r0, TAIL)], tidx_v)
        pltpu.async_copy(nodes.at[pl.ds(r0, TAIL)], tail_v, sem_t).wait()
        pltpu.sync_copy(tail_v, acc_sh.at[tidx_v], add=True)

    plsc.subcore_barrier()

    @pl.when(s == 0)
    def _flush():
        pltpu.sync_copy(acc_sh, out.at[c])


@jax.jit
def _segsum(nodes, ids32, zeros):
    mesh = plsc.VectorSubcoreMesh(core_axis_name="c", subcore_axis_name="s")
    partials = pl.kernel(
        _body,
        out_type=jax.ShapeDtypeStruct((NC, NSEG, D), jnp.float32),
        mesh=mesh,
        scratch_types=[
            [pltpu.VMEM((CHUNK,), jnp.int32) for _ in range(NBUF)],
            [pltpu.VMEM((CHUNK, D), jnp.float32) for _ in range(NBUF)],
            pltpu.VMEM((TAIL,), jnp.int32),
            pltpu.VMEM((TAIL, D), jnp.float32),
            pltpu.VMEM_SHARED((NSEG, D), jnp.float32),
            [pltpu.SemaphoreType.DMA for _ in range(NBUF)],
            [pltpu.SemaphoreType.DMA for _ in range(NBUF)],
            [pltpu.SemaphoreType.DMA for _ in range(NBUF)],
            pltpu.SemaphoreType.DMA,
        ],
    )(nodes, ids32, zeros)
    return partials[0] + partials[1]


def kernel(nodes, segment_ids, num_graphs):
    del num_graphs
    ids32 = segment_ids.astype(jnp.int32)
    zeros = jnp.zeros((NSEG, D), jnp.float32)
    return _segsum(nodes, ids32, zeros)


# speedup vs baseline: 1.4566x; 1.0086x over previous
import jax
import jax.numpy as jnp
from jax import lax
from jax.experimental import pallas as pl
from jax.experimental.pallas import tpu as pltpu
from jax.experimental.pallas import tpu_sc as plsc

N_ROWS = 100000
D = 128
NSEG = 64
CHUNK = 128
N_FULL = N_ROWS // CHUNK
TAIL = N_ROWS - N_FULL * CHUNK
NC, NS = 2, 16
NW = NC * NS
MAXC = -(-N_FULL // NW)
HI = N_FULL - (MAXC - 1) * NW
NBUF = 7


def _body(nodes, ids, zeros, out,
          ibufs, bufs, tidx_v, tail_v, acc_sh, semns, semis, semscs, sem_t):
    c = lax.axis_index("c")
    s = lax.axis_index("s")
    wid = s * NC + c

    def gather(j):
        b = j % NBUF
        r0 = (wid + j * NW) * CHUNK
        pltpu.async_copy(nodes.at[pl.ds(r0, CHUNK)], bufs[b], semns[b])
        pltpu.async_copy(ids.at[pl.ds(r0, CHUNK)], ibufs[b], semis[b])

    def gather_wait(j):
        b = j % NBUF
        pltpu.make_async_copy(nodes.at[pl.ds(0, CHUNK)], bufs[b], semns[b]).wait()
        pltpu.make_async_copy(ids.at[pl.ds(0, CHUNK)], ibufs[b], semis[b]).wait()

    def scatter(j):
        b = j % NBUF
        pltpu.async_copy(bufs[b], acc_sh.at[ibufs[b]], semscs[b], add=True)

    def scatter_wait(j):
        b = j % NBUF
        pltpu.make_async_copy(bufs[b], acc_sh.at[ibufs[b]], semscs[b]).wait()

    for j0 in range(NBUF - 2):
        gather(j0)

    @pl.when(s == 0)
    def _init():
        pltpu.sync_copy(zeros, acc_sh)

    plsc.subcore_barrier()

    for i in range(MAXC):
        if i >= 2:
            scatter_wait(i - 2)

        def step(i=i):
            j = i + NBUF - 2
            if j < MAXC:
                if j == MAXC - 1:
                    @pl.when(wid < HI)
                    def _():
                        gather(j)
                else:
                    gather(j)
            gather_wait(i)
            scatter(i)

        if i == MAXC - 1:
            @pl.when(wid < HI)
            def _():
                step()
        else:
            step()

    scatter_wait(MAXC - 2)

    @pl.when(wid < HI)
    def _last():
        scatter_wait(MAXC - 1)

    @pl.when(wid == NW - 1)
    def _tail():
        r0 = N_FULL * CHUNK
        pltpu.sync_copy(ids.at[pl.ds(r0, TAIL)], tidx_v)
        pltpu.async_copy(nodes.at[pl.ds(r0, TAIL)], tail_v, sem_t).wait()
        pltpu.sync_copy(tail_v, acc_sh.at[tidx_v], add=True)

    plsc.subcore_barrier()

    @pl.when(s == 0)
    def _flush():
        pltpu.sync_copy(acc_sh, out.at[c])


@jax.jit
def _segsum(nodes, ids32, zeros):
    mesh = plsc.VectorSubcoreMesh(core_axis_name="c", subcore_axis_name="s")
    partials = pl.kernel(
        _body,
        out_type=jax.ShapeDtypeStruct((NC, NSEG, D), jnp.float32),
        mesh=mesh,
        scratch_types=[
            [pltpu.VMEM((CHUNK,), jnp.int32) for _ in range(NBUF)],
            [pltpu.VMEM((CHUNK, D), jnp.float32) for _ in range(NBUF)],
            pltpu.VMEM((TAIL,), jnp.int32),
            pltpu.VMEM((TAIL, D), jnp.float32),
            pltpu.VMEM_SHARED((NSEG, D), jnp.float32),
            [pltpu.SemaphoreType.DMA for _ in range(NBUF)],
            [pltpu.SemaphoreType.DMA for _ in range(NBUF)],
            [pltpu.SemaphoreType.DMA for _ in range(NBUF)],
            pltpu.SemaphoreType.DMA,
        ],
    )(nodes, ids32, zeros)
    return partials[0] + partials[1]


def kernel(nodes, segment_ids, num_graphs):
    del num_graphs
    ids32 = segment_ids.astype(jnp.int32)
    zeros = jnp.zeros((NSEG, D), jnp.float32)
    return _segsum(nodes, ids32, zeros)
